# baseline (speedup 1.0000x reference)
import functools

import jax
import jax.numpy as jnp
from jax import lax
from jax.experimental import pallas as pl
from jax.experimental.pallas import tpu as pltpu
from jax.experimental.pallas import tpu_sc as plsc

N_ROWS = 1_000_000
WSC = 8_192
_NC, _NS = 2, 16
NW = _NC * _NS
S = NW * WSC
BT = 8_192
TC_OFF = S // BT
TC_GRID = -(-(N_ROWS - S) // BT)


def _sc_body(x_hbm, m_hbm, o_hbm, xv, yv, mv, si, so):
    wid = lax.axis_index("s") * _NC + lax.axis_index("c")
    base = wid * WSC
    h_in = pltpu.async_copy(x_hbm.at[:, pl.ds(base, WSC)], xv, si)
    pltpu.sync_copy(m_hbm, mv)
    mvec = mv[...]
    m = [mvec[i] for i in range(9)]
    h_in.wait()

    @plsc.parallel_loop(0, WSC, step=16, unroll=8)
    def _(s):
        x0 = xv[0, pl.ds(s, 16)]
        x1 = xv[1, pl.ds(s, 16)]
        x2 = xv[2, pl.ds(s, 16)]
        yv[0, pl.ds(s, 16)] = m[0] * x0 + m[1] * x1 + m[2] * x2
        yv[1, pl.ds(s, 16)] = m[3] * x0 + m[4] * x1 + m[5] * x2
        yv[2, pl.ds(s, 16)] = m[6] * x0 + m[7] * x1 + m[8] * x2

    pltpu.async_copy(yv, o_hbm.at[:, pl.ds(base, WSC)], so).wait()


_sc_call = functools.partial(
    pl.kernel,
    out_type=jax.ShapeDtypeStruct((3, S), jnp.float32),
    mesh=plsc.VectorSubcoreMesh(core_axis_name="c", subcore_axis_name="s"),
    scratch_types=[
        pltpu.VMEM((3, WSC), jnp.float32),
        pltpu.VMEM((3, WSC), jnp.float32),
        pltpu.VMEM((16,), jnp.float32),
        pltpu.SemaphoreType.DMA,
        pltpu.SemaphoreType.DMA,
    ],
    compiler_params=pltpu.CompilerParams(needs_layout_passes=False),
)(_sc_body)


def _tc_body(m_ref, x_ref, o_ref):
    x = x_ref[...]
    m = m_ref
    y0 = m[0, 0] * x[0:1, :] + m[0, 1] * x[1:2, :] + m[0, 2] * x[2:3, :]
    y1 = m[1, 0] * x[0:1, :] + m[1, 1] * x[1:2, :] + m[1, 2] * x[2:3, :]
    y2 = m[2, 0] * x[0:1, :] + m[2, 1] * x[1:2, :] + m[2, 2] * x[2:3, :]
    o_ref[...] = jnp.concatenate([y0, y1, y2], axis=0)


def _tc_call(xt, mat):
    return pl.pallas_call(
        _tc_body,
        grid=(TC_GRID,),
        in_specs=[
            pl.BlockSpec(memory_space=pltpu.SMEM),
            pl.BlockSpec((3, BT), lambda i: (0, i + TC_OFF)),
        ],
        out_specs=pl.BlockSpec((3, BT), lambda i: (0, i + TC_OFF)),
        out_shape=jax.ShapeDtypeStruct((3, N_ROWS), jnp.float32),
    )(mat, xt)


def kernel(sp_mat, mat):
    xt = sp_mat.T
    m16 = jnp.zeros((16,), jnp.float32).at[:9].set(mat.reshape(-1))
    sc_out = _sc_call(xt, m16)
    tc_out = _tc_call(xt, mat)
    out_t = lax.dynamic_update_slice(tc_out, sc_out, (0, 0))
    return out_t.T


# speedup vs baseline: 1.7916x; 1.7916x over previous
import functools

import jax
import jax.numpy as jnp
from jax import lax
from jax.experimental import pallas as pl
from jax.experimental.pallas import tpu as pltpu
from jax.experimental.pallas import tpu_sc as plsc

N_ROWS = 1_000_000
WSC = 8_192
_NC, _NS = 2, 16
NW = _NC * _NS
S = NW * WSC
BT = 32_768
TC_OFF = S // BT
TC_GRID = -(-(N_ROWS - S) // BT)


def _sc_body(x_hbm, m_hbm, o_hbm, xv, yv, mv, si, so):
    wid = lax.axis_index("s") * _NC + lax.axis_index("c")
    base = wid * WSC
    h_in = pltpu.async_copy(x_hbm.at[:, pl.ds(base, WSC)], xv, si)
    pltpu.sync_copy(m_hbm, mv)
    mvec = mv[...]
    m = [mvec[i] for i in range(9)]
    h_in.wait()

    @plsc.parallel_loop(0, WSC, step=16, unroll=8)
    def _(s):
        x0 = xv[0, pl.ds(s, 16)]
        x1 = xv[1, pl.ds(s, 16)]
        x2 = xv[2, pl.ds(s, 16)]
        yv[0, pl.ds(s, 16)] = m[0] * x0 + m[1] * x1 + m[2] * x2
        yv[1, pl.ds(s, 16)] = m[3] * x0 + m[4] * x1 + m[5] * x2
        yv[2, pl.ds(s, 16)] = m[6] * x0 + m[7] * x1 + m[8] * x2

    pltpu.async_copy(yv, o_hbm.at[:, pl.ds(base, WSC)], so).wait()


_sc_call = functools.partial(
    pl.kernel,
    out_type=jax.ShapeDtypeStruct((3, S), jnp.float32),
    mesh=plsc.VectorSubcoreMesh(core_axis_name="c", subcore_axis_name="s"),
    scratch_types=[
        pltpu.VMEM((3, WSC), jnp.float32),
        pltpu.VMEM((3, WSC), jnp.float32),
        pltpu.VMEM((16,), jnp.float32),
        pltpu.SemaphoreType.DMA,
        pltpu.SemaphoreType.DMA,
    ],
    compiler_params=pltpu.CompilerParams(needs_layout_passes=False),
)(_sc_body)


def _tc_body(m_ref, x_ref, o_ref):
    o_ref[...] = lax.dot_general(
        m_ref[...], x_ref[...], (((1,), (0,)), ((), ())),
        preferred_element_type=jnp.float32)


def _tc_call(xt, mat):
    return pl.pallas_call(
        _tc_body,
        grid=(TC_GRID,),
        in_specs=[
            pl.BlockSpec((3, 3), lambda i: (0, 0)),
            pl.BlockSpec((3, BT), lambda i: (0, i + TC_OFF)),
        ],
        out_specs=pl.BlockSpec((3, BT), lambda i: (0, i + TC_OFF)),
        out_shape=jax.ShapeDtypeStruct((3, N_ROWS), jnp.float32),
    )(mat, xt)


def kernel(sp_mat, mat):
    xt = sp_mat.T
    m16 = jnp.zeros((16,), jnp.float32).at[:9].set(mat.reshape(-1))
    sc_out = _sc_call(xt, m16)
    tc_out = _tc_call(xt, mat)
    out_t = lax.dynamic_update_slice(tc_out, sc_out, (0, 0))
    return out_t.T


# speedup vs baseline: 1.9690x; 1.0990x over previous
import functools

import jax
import jax.numpy as jnp
from jax import lax
from jax.experimental import pallas as pl
from jax.experimental.pallas import tpu as pltpu
from jax.experimental.pallas import tpu_sc as plsc

N_ROWS = 1_000_000
WSC = 12_288
_NC, _NS = 2, 16
NW = _NC * _NS
S = NW * WSC
BT = 65_536
TC_OFF = S // BT
TC_GRID = -(-(N_ROWS - S) // BT)


def _sc_body(x_hbm, m_hbm, o_hbm, xv, yv, mv, si, so):
    wid = lax.axis_index("s") * _NC + lax.axis_index("c")
    base = wid * WSC
    h_in = pltpu.async_copy(x_hbm.at[:, pl.ds(base, WSC)], xv, si)
    pltpu.sync_copy(m_hbm, mv)
    mvec = mv[...]
    m = [mvec[i] for i in range(9)]
    h_in.wait()

    @plsc.parallel_loop(0, WSC, step=16, unroll=8)
    def _(s):
        x0 = xv[0, pl.ds(s, 16)]
        x1 = xv[1, pl.ds(s, 16)]
        x2 = xv[2, pl.ds(s, 16)]
        yv[0, pl.ds(s, 16)] = m[0] * x0 + m[1] * x1 + m[2] * x2
        yv[1, pl.ds(s, 16)] = m[3] * x0 + m[4] * x1 + m[5] * x2
        yv[2, pl.ds(s, 16)] = m[6] * x0 + m[7] * x1 + m[8] * x2

    pltpu.async_copy(yv, o_hbm.at[:, pl.ds(base, WSC)], so).wait()


_sc_call = functools.partial(
    pl.kernel,
    out_type=jax.ShapeDtypeStruct((3, S), jnp.float32),
    mesh=plsc.VectorSubcoreMesh(core_axis_name="c", subcore_axis_name="s"),
    scratch_types=[
        pltpu.VMEM((3, WSC), jnp.float32),
        pltpu.VMEM((3, WSC), jnp.float32),
        pltpu.VMEM((16,), jnp.float32),
        pltpu.SemaphoreType.DMA,
        pltpu.SemaphoreType.DMA,
    ],
    compiler_params=pltpu.CompilerParams(needs_layout_passes=False),
)(_sc_body)


def _tc_body(m_ref, x_ref, o_ref):
    o_ref[...] = lax.dot_general(
        m_ref[...], x_ref[...], (((1,), (0,)), ((), ())),
        preferred_element_type=jnp.float32)


def _tc_call(xt, mat):
    return pl.pallas_call(
        _tc_body,
        grid=(TC_GRID,),
        in_specs=[
            pl.BlockSpec((3, 3), lambda i: (0, 0)),
            pl.BlockSpec((3, BT), lambda i: (0, i + TC_OFF)),
        ],
        out_specs=pl.BlockSpec((3, BT), lambda i: (0, i + TC_OFF)),
        out_shape=jax.ShapeDtypeStruct((3, N_ROWS), jnp.float32),
    )(mat, xt)


def kernel(sp_mat, mat):
    xt = sp_mat.T
    m16 = jnp.zeros((16,), jnp.float32).at[:9].set(mat.reshape(-1))
    sc_out = _sc_call(xt, m16)
    tc_out = _tc_call(xt, mat)
    out_t = lax.dynamic_update_slice(tc_out, sc_out, (0, 0))
    return out_t.T


# speedup vs baseline: 1.9901x; 1.0107x over previous
import functools

import jax
import jax.numpy as jnp
from jax import lax
from jax.experimental import pallas as pl
from jax.experimental.pallas import tpu as pltpu
from jax.experimental.pallas import tpu_sc as plsc

N_ROWS = 1_000_000
WSC = 12_288
_NC, _NS = 2, 16
NW = _NC * _NS
S = NW * WSC
BT = 131_072
TC_OFF = S // BT
TC_GRID = -(-(N_ROWS - S) // BT)


def _sc_body(x_hbm, m_hbm, o_hbm, xv, yv, mv, si, so):
    wid = lax.axis_index("s") * _NC + lax.axis_index("c")
    base = wid * WSC
    h_in = pltpu.async_copy(x_hbm.at[:, pl.ds(base, WSC)], xv, si)
    pltpu.sync_copy(m_hbm, mv)
    mvec = mv[...]
    m = [mvec[i] for i in range(9)]
    h_in.wait()

    @plsc.parallel_loop(0, WSC, step=16, unroll=8)
    def _(s):
        x0 = xv[0, pl.ds(s, 16)]
        x1 = xv[1, pl.ds(s, 16)]
        x2 = xv[2, pl.ds(s, 16)]
        yv[0, pl.ds(s, 16)] = m[0] * x0 + m[1] * x1 + m[2] * x2
        yv[1, pl.ds(s, 16)] = m[3] * x0 + m[4] * x1 + m[5] * x2
        yv[2, pl.ds(s, 16)] = m[6] * x0 + m[7] * x1 + m[8] * x2

    pltpu.async_copy(yv, o_hbm.at[:, pl.ds(base, WSC)], so).wait()


_sc_call = functools.partial(
    pl.kernel,
    out_type=jax.ShapeDtypeStruct((3, S), jnp.float32),
    mesh=plsc.VectorSubcoreMesh(core_axis_name="c", subcore_axis_name="s"),
    scratch_types=[
        pltpu.VMEM((3, WSC), jnp.float32),
        pltpu.VMEM((3, WSC), jnp.float32),
        pltpu.VMEM((16,), jnp.float32),
        pltpu.SemaphoreType.DMA,
        pltpu.SemaphoreType.DMA,
    ],
    compiler_params=pltpu.CompilerParams(needs_layout_passes=False),
)(_sc_body)


def _tc_body(m_ref, x_ref, o_ref):
    o_ref[...] = lax.dot_general(
        m_ref[...], x_ref[...], (((1,), (0,)), ((), ())),
        preferred_element_type=jnp.float32)


def _tc_call(xt, mat):
    return pl.pallas_call(
        _tc_body,
        grid=(TC_GRID,),
        in_specs=[
            pl.BlockSpec((3, 3), lambda i: (0, 0)),
            pl.BlockSpec((3, BT), lambda i: (0, i + TC_OFF)),
        ],
        out_specs=pl.BlockSpec((3, BT), lambda i: (0, i + TC_OFF)),
        out_shape=jax.ShapeDtypeStruct((3, N_ROWS), jnp.float32),
    )(mat, xt)


def kernel(sp_mat, mat):
    xt = sp_mat.T
    m16 = jnp.zeros((16,), jnp.float32).at[:9].set(mat.reshape(-1))
    sc_out = _sc_call(xt, m16)
    tc_out = _tc_call(xt, mat)
    out_t = lax.dynamic_update_slice(tc_out, sc_out, (0, 0))
    return out_t.T
